# baseline (speedup 1.0000x reference)
import functools

import jax
import jax.numpy as jnp
from jax import lax
from jax.experimental import pallas as pl
from jax.experimental.pallas import tpu as pltpu
from jax.experimental.pallas import tpu_sc as plsc

DIM = 128
BATCH = 16384
LANES = 16
CHUNK = 64


def _hour_from_unix(tv):
    n = lax.shift_right_logical(tv, 7)
    q = (n.astype(jnp.float32) * jnp.float32(1.0 / 675.0)).astype(jnp.int32)
    r = tv - q * 86400
    r = jnp.where(r < 0, r + 86400, r)
    r = jnp.where(r >= 86400, r - 86400, r)
    h = (r.astype(jnp.float32) * jnp.float32(1.0 / 3600.0)).astype(jnp.int32)
    rem = r - h * 3600
    h = jnp.where(rem < 0, h - 1, h)
    rem = jnp.where(rem < 0, rem + 3600, rem)
    h = jnp.where(rem >= 3600, h + 1, h)
    return h


def kernel(t, week_emb, day_emb, month_emb, hour_emb):
    del week_emb, day_emb, month_emb
    info = plsc.get_sparse_core_info()
    nc, ns = info.num_cores, info.num_subcores
    nw = nc * ns
    bpw = BATCH // nw
    nchunks = bpw // CHUNK

    mesh = plsc.VectorSubcoreMesh(core_axis_name="c", subcore_axis_name="s")

    @functools.partial(
        pl.kernel,
        mesh=mesh,
        out_type=jax.ShapeDtypeStruct((BATCH, DIM), jnp.float32),
        scratch_types=[
            pltpu.VMEM((bpw,), jnp.int32),
            pltpu.VMEM((nchunks, CHUNK), jnp.int32),
            pltpu.VMEM((nchunks, CHUNK, DIM), jnp.float32),
            pltpu.VMEM_SHARED((24, DIM), jnp.float32),
        ] + [pltpu.SemaphoreType.DMA] * (nchunks + 3),
    )
    def sc_lookup(t_hbm, tab_hbm, out_hbm, t_v, idx_v, rows_v, tab_sh, *sems):
        gsems, ssem, stsem = sems[:nchunks + 1], sems[nchunks + 1], sems[nchunks + 2]
        sid = lax.axis_index("s")
        wid = sid * nc + lax.axis_index("c")
        base = wid * bpw

        @pl.when(sid == 0)
        def _stage_table():
            pltpu.async_copy(tab_hbm, tab_sh, stsem)

        pltpu.sync_copy(t_hbm.at[pl.ds(base, bpw)], t_v)

        @pl.when(sid == 0)
        def _stage_wait():
            pltpu.make_async_copy(tab_hbm, tab_sh, stsem).wait()

        pieces = [(0, LANES), (LANES, CHUNK - LANES)]
        pieces += [(j * CHUNK, CHUNK) for j in range(1, nchunks)]

        gathers = []
        for p, (off, ln) in enumerate(pieces):
            row, col = off // CHUNK, off % CHUNK

            def _compute(k, carry, off=off, row=row, col=col):
                tv = t_v[pl.ds(off + k * LANES, LANES)]
                idx_v[row, pl.ds(col + k * LANES, LANES)] = _hour_from_unix(tv)
                return carry

            lax.fori_loop(0, ln // LANES, _compute, 0)
            if p == 0:
                plsc.subcore_barrier()
            gathers.append(
                pltpu.async_copy(tab_sh.at[idx_v.at[row, pl.ds(col, ln)]],
                                 rows_v.at[row, pl.ds(col, ln)], gsems[p]))

        scatters = []
        for p, (off, ln) in enumerate(pieces):
            row, col = off // CHUNK, off % CHUNK
            gathers[p].wait()
            scatters.append(
                pltpu.async_copy(rows_v.at[row, pl.ds(col, ln)],
                                 out_hbm.at[pl.ds(base + off, ln)], ssem))
        for p in range(len(pieces)):
            scatters[p].wait()

    return sc_lookup(t, hour_emb)
